# Initial kernel scaffold; baseline (speedup 1.0000x reference)
import jax
import jax.numpy as jnp
from jax.experimental import pallas as pl


def kernel(x, edge_index, edge_attr, W_e, b_e, W1, b1, W2, b2, gamma, beta):
    raise NotImplementedError("write your pallas kernel here")


# speedup vs baseline: 2.1681x; 2.1681x over previous
import functools

import jax
import jax.numpy as jnp
from jax import lax
from jax.experimental import pallas as pl
from jax.experimental.pallas import tpu as pltpu
from jax.experimental.pallas import tpu_sc as plsc

NC = 2
NS = 16
LANES = 16


def _eproj_body(ea_ref, we_ref, be_ref, out_ref):
    out_ref[...] = (
        jnp.dot(ea_ref[...], we_ref[...], preferred_element_type=jnp.float32)
        + be_ref[...]
    )


def _edge_proj(edge_attr, W_e, b_e):
    E, DE = edge_attr.shape
    D = W_e.shape[1]
    BE = 1280
    return pl.pallas_call(
        _eproj_body,
        grid=(E // BE,),
        in_specs=[
            pl.BlockSpec((BE, DE), lambda i: (i, 0)),
            pl.BlockSpec((DE, D), lambda i: (0, 0)),
            pl.BlockSpec((1, D), lambda i: (0, 0)),
        ],
        out_specs=pl.BlockSpec((BE, D), lambda i: (i, 0)),
        out_shape=jax.ShapeDtypeStruct((E, D), jnp.float32),
    )(edge_attr, W_e, b_e.reshape(1, D))


def _make_sc_aggregate(NP, E, D):
    EPW = E // (NC * NS)
    C = 80
    NCHUNK = EPW // C
    RPT = NP // NS
    ZR = 128
    assert EPW % C == 0 and RPT % ZR == 0 and D % LANES == 0 and RPT % 8 == 0
    mesh = plsc.VectorSubcoreMesh(core_axis_name="c", subcore_axis_name="s")

    @functools.partial(
        pl.kernel,
        out_type=jax.ShapeDtypeStruct((NC, NP, D), jnp.float32),
        mesh=mesh,
        scratch_types=[
            pltpu.VMEM((C,), jnp.int32),
            pltpu.VMEM((C,), jnp.int32),
            pltpu.VMEM((C, D), jnp.float32),
            pltpu.VMEM((C, D), jnp.float32),
            pltpu.VMEM((ZR, D), jnp.float32),
            pltpu.VMEM_SHARED((NP, D), jnp.float32),
            pltpu.SemaphoreType.DMA,
        ],
    )
    def sc_aggregate(x_hbm, e_hbm, src_hbm, dst_hbm, out_hbm,
                     sidx, didx, xbuf, ebuf, zbuf, aggr, sem):
        cid = lax.axis_index("c")
        sid = lax.axis_index("s")

        zero = jnp.zeros((LANES,), jnp.float32)

        def zrow(i, _):
            for j in range(D // LANES):
                zbuf[i, pl.ds(j * LANES, LANES)] = zero
            return 0

        lax.fori_loop(0, ZR, zrow, 0)
        for k in range(RPT // ZR):
            pltpu.sync_copy(zbuf, aggr.at[pl.ds(sid * RPT + k * ZR, ZR)])
        plsc.subcore_barrier()

        base = (cid * NS + sid) * EPW

        def chunk(i, _):
            off = base + i * C
            pltpu.sync_copy(src_hbm.at[pl.ds(off, C)], sidx)
            pltpu.sync_copy(dst_hbm.at[pl.ds(off, C)], didx)
            pltpu.sync_copy(e_hbm.at[pl.ds(off, C)], ebuf)
            pltpu.async_copy(x_hbm.at[sidx], xbuf, sem).wait()

            def row(r, _):
                for j in range(D // LANES):
                    sl = pl.ds(j * LANES, LANES)
                    ebuf[r, sl] = jnp.maximum(
                        xbuf[r, sl] + ebuf[r, sl], 0.0)
                return 0

            lax.fori_loop(0, C, row, 0)
            pltpu.sync_copy(ebuf, aggr.at[didx], add=True)
            return 0

        lax.fori_loop(0, NCHUNK, chunk, 0)
        plsc.subcore_barrier()
        pltpu.sync_copy(
            aggr.at[pl.ds(sid * RPT, RPT)],
            out_hbm.at[cid, pl.ds(sid * RPT, RPT)],
        )

    return sc_aggregate


def _node_body(x_ref, p_ref, w1_ref, b1_ref, w2_ref, b2_ref, g_ref, bt_ref,
               out_ref):
    h = x_ref[...] + p_ref[0] + p_ref[1]
    h = jnp.maximum(
        jnp.dot(h, w1_ref[...], preferred_element_type=jnp.float32)
        + b1_ref[...], 0.0)
    h = jnp.dot(h, w2_ref[...], preferred_element_type=jnp.float32) + b2_ref[...]
    mean = jnp.mean(h, axis=0, keepdims=True)
    var = jnp.mean((h - mean) ** 2, axis=0, keepdims=True)
    h = g_ref[...] * (h - mean) * lax.rsqrt(var + 1e-5) + bt_ref[...]
    out_ref[...] = jnp.maximum(h, 0.0)


def _node_mlp(x, partials, W1, b1, W2, b2, gamma, beta):
    N, D = x.shape
    return pl.pallas_call(
        _node_body,
        out_shape=jax.ShapeDtypeStruct((N, D), jnp.float32),
    )(x, partials, W1, b1.reshape(1, D), W2, b2.reshape(1, D),
      gamma.reshape(1, D), beta.reshape(1, D))


def kernel(x, edge_index, edge_attr, W_e, b_e, W1, b1, W2, b2, gamma, beta):
    N, D = x.shape
    E = edge_attr.shape[0]
    src = edge_index[0].astype(jnp.int32)
    dst = edge_index[1].astype(jnp.int32)
    e = _edge_proj(edge_attr, W_e, b_e)
    NP = ((N + NS * 128 - 1) // (NS * 128)) * (NS * 128)
    partials = _make_sc_aggregate(NP, E, D)(x, e, src, dst)
    return _node_mlp(x, partials[:, :N], W1, b1, W2, b2, gamma, beta)
